# baseline (speedup 1.0000x reference)
import functools

import jax
import jax.numpy as jnp
from jax import lax
from jax.experimental import pallas as pl
from jax.experimental.pallas import tpu as pltpu
from jax.experimental.pallas import tpu_sc as plsc

N_NODES = 10000
ROWS = 10240
C = 128
NC = 2
NS = 16


def _tc_matmul_split(x, w, b):
    R, K = x.shape
    M = w.shape[1]
    RB = R // 8

    def body(x_ref, w_ref, b_ref, o_ref):
        r = (jnp.dot(x_ref[...], w_ref[...], preferred_element_type=jnp.float32)
             + b_ref[...])
        o_ref[0] = r[:, :64]
        o_ref[1] = r[:, 64:]

    return pl.pallas_call(
        body,
        grid=(8,),
        in_specs=[
            pl.BlockSpec((RB, K), lambda i: (i, 0)),
            pl.BlockSpec((K, M), lambda i: (0, 0)),
            pl.BlockSpec((1, M), lambda i: (0, 0)),
        ],
        out_specs=pl.BlockSpec((2, RB, 64), lambda i: (0, i, 0)),
        out_shape=jax.ShapeDtypeStruct((2, R, 64), jnp.float32),
    )(x, w, b.reshape(1, M))


def _tc_combine1(acc, bias1, rmat, w2, b2):
    RB = ROWS // 8

    def body(a_ref, b1_ref, r_ref, w2_ref, b2_ref, o_ref):
        msg = jnp.concatenate([a_ref[0, :, :64], a_ref[1, :, :64]], axis=1)
        den = jnp.concatenate([a_ref[0, :, 64:68], a_ref[1, :, 64:68]], axis=1)
        denx = jnp.dot(den, r_ref[...], preferred_element_type=jnp.float32) + 1e-16
        h = jnp.maximum(msg / denx + b1_ref[...], 0.0)
        o_ref[...] = (
            jnp.dot(h, w2_ref[...], preferred_element_type=jnp.float32) + b2_ref[...]
        )

    return pl.pallas_call(
        body,
        grid=(8,),
        in_specs=[
            pl.BlockSpec((2, RB, 80), lambda i: (0, i, 0)),
            pl.BlockSpec((1, 128), lambda i: (0, 0)),
            pl.BlockSpec((8, 128), lambda i: (0, 0)),
            pl.BlockSpec((128, 64), lambda i: (0, 0)),
            pl.BlockSpec((1, 64), lambda i: (0, 0)),
        ],
        out_specs=pl.BlockSpec((RB, 64), lambda i: (i, 0)),
        out_shape=jax.ShapeDtypeStruct((ROWS, 64), jnp.float32),
    )(acc, bias1.reshape(1, 128), rmat, w2, b2.reshape(1, 64))


def _tc_final(acc, bias2):
    RB = ROWS // 8

    def body(a_ref, b2_ref, o_ref):
        a = a_ref[0] + a_ref[1]
        msg = a[:, :64]
        den = a[:, 64:65]
        o = msg / (den + 1e-16) + b2_ref[...]
        m = jnp.max(o, axis=1, keepdims=True)
        ls = m + jnp.log(jnp.sum(jnp.exp(o - m), axis=1, keepdims=True))
        o_ref[...] = o - ls

    return pl.pallas_call(
        body,
        grid=(8,),
        in_specs=[
            pl.BlockSpec((2, RB, 80), lambda i: (0, i, 0)),
            pl.BlockSpec((1, 64), lambda i: (0, 0)),
        ],
        out_specs=pl.BlockSpec((RB, 64), lambda i: (i, 0)),
        out_shape=jax.ShapeDtypeStruct((ROWS, 64), jnp.float32),
    )(acc, bias2.reshape(1, 64))


def _sc_edge_pass(heads, grp, e_pad, head_split):
    CH = heads * grp * 16
    W = CH + 16
    n_workers = NS if head_split else NC * NS
    EPT = e_pad // n_workers
    NCHUNK = EPT // C
    RPT = ROWS // NS
    mesh = plsc.VectorSubcoreMesh(core_axis_name="c", subcore_axis_name="s")

    @functools.partial(
        pl.kernel,
        out_type=jax.ShapeDtypeStruct((NC, ROWS, W), jnp.float32),
        mesh=mesh,
        compiler_params=pltpu.CompilerParams(
            use_tc_tiling_on_sc=False, needs_layout_passes=False),
        scratch_types=[
            pltpu.VMEM((C,), jnp.int32),
            pltpu.VMEM((3, C), jnp.int32),
            pltpu.VMEM((2, C + 16), jnp.float32),
            pltpu.VMEM((2, C, CH), jnp.float32),
            pltpu.VMEM((2, C, CH), jnp.float32),
            pltpu.VMEM((2, C, W), jnp.float32),
            pltpu.VMEM((heads * grp, 16), jnp.float32),
            pltpu.VMEM((heads * grp, 16), jnp.float32),
            pltpu.VMEM_SHARED((ROWS, W), jnp.float32),
            pltpu.SemaphoreType.DMA,
            pltpu.SemaphoreType.DMA,
            pltpu.SemaphoreType.DMA,
            pltpu.SemaphoreType.DMA,
        ],
    )
    def k(src_hbm, dst_hbm, ea_hbm, xl_hbm, att_hbm, we_hbm, init_hbm, out_hbm,
          src_v, dst_v, ea_v, xs_v, xd_v, msg_v, att_v, we_v, acc_sh,
          sem1, sem2, sem3, sem4):
        cid = lax.axis_index("c")
        sid = lax.axis_index("s")
        if head_split:
            wid = sid
            tbl = xl_hbm.at[cid]
            pltpu.sync_copy(att_hbm.at[cid], att_v)
            pltpu.sync_copy(we_hbm.at[cid], we_v)
        else:
            wid = cid * NS + sid
            tbl = xl_hbm
            pltpu.sync_copy(att_hbm, att_v)
            pltpu.sync_copy(we_hbm, we_v)
        pltpu.sync_copy(init_hbm.at[pl.ds(sid * RPT, RPT)],
                        acc_sh.at[pl.ds(sid * RPT, RPT)])
        lane = lax.iota(jnp.int32, 16)
        rots = [(lane + s) % 16 for s in (8, 4, 2, 1)]
        att_vals = [att_v[i, :] for i in range(heads * grp)]
        we_vals = [we_v[i, :] for i in range(heads * grp)]
        plsc.subcore_barrier()
        base_w = wid * EPT

        def chunk_body(kk, carry):
            q = kk % 2
            p = 1 - q
            r3 = lax.rem(kk, 3)
            r3p = lax.rem(kk + 2, 3)
            cps = []

            @pl.when(jnp.logical_and(kk >= 3, p == 0))
            def _wait_scatter0():
                pltpu.make_async_copy(
                    msg_v.at[0], acc_sh.at[dst_v.at[r3]], sem3).wait()

            @pl.when(jnp.logical_and(kk >= 3, p == 1))
            def _wait_scatter1():
                pltpu.make_async_copy(
                    msg_v.at[1], acc_sh.at[dst_v.at[r3]], sem4).wait()

            @pl.when(kk < NCHUNK)
            def _issue():
                base = base_w + kk * C
                pltpu.sync_copy(src_hbm.at[pl.ds(base, C)], src_v)
                pltpu.sync_copy(dst_hbm.at[pl.ds(base, C)], dst_v.at[r3])
                pltpu.sync_copy(ea_hbm.at[pl.ds(base, C)],
                                ea_v.at[q, pl.ds(0, C)])
                cps.append(pltpu.async_copy(tbl.at[src_v], xs_v.at[q], sem1))
                cps.append(pltpu.async_copy(tbl.at[dst_v.at[r3]], xd_v.at[q], sem2))

            @pl.when(kk > 0)
            def _compute():
                @plsc.parallel_loop(0, C, step=1, unroll=6)
                def edge_body(e):
                    ea_s = ea_v[p, pl.ds(e, 16)][0]
                    den = jnp.zeros((16,), jnp.float32)
                    for h in range(heads):
                        t = jnp.zeros((16,), jnp.float32)
                        xsl = []
                        for g in range(grp):
                            j = (h * grp + g) * 16
                            xs = xs_v[p, e, pl.ds(j, 16)]
                            xd = xd_v[p, e, pl.ds(j, 16)]
                            m = xs + xd + ea_s * we_vals[h * grp + g]
                            m = jnp.maximum(m, 0.2 * m)
                            t = t + m * att_vals[h * grp + g]
                            xsl.append(xs)
                        pv = jnp.exp(
                            jnp.full((16,), plsc.cumsum(t)[15], jnp.float32))
                        for g in range(grp):
                            msg_v[p, e, pl.ds((h * grp + g) * 16, 16)] = xsl[g] * pv
                        den = den + jnp.where(lane == h, pv, 0.0)
                    msg_v[p, e, pl.ds(CH, 16)] = den

                @pl.when(p == 0)
                def _scatter0():
                    pltpu.async_copy(
                        msg_v.at[0], acc_sh.at[dst_v.at[r3p]], sem3, add=True)

                @pl.when(p == 1)
                def _scatter1():
                    pltpu.async_copy(
                        msg_v.at[1], acc_sh.at[dst_v.at[r3p]], sem4, add=True)

            @pl.when(kk < NCHUNK)
            def _drain():
                for cp in cps:
                    cp.wait()

            return carry

        lax.fori_loop(0, NCHUNK + 1, chunk_body, 0)
        for chunk in (NCHUNK - 2, NCHUNK - 1):
            pltpu.make_async_copy(
                msg_v.at[chunk % 2],
                acc_sh.at[dst_v.at[chunk % 3]],
                sem3 if chunk % 2 == 0 else sem4).wait()
        plsc.subcore_barrier()
        pltpu.sync_copy(acc_sh.at[pl.ds(sid * RPT, RPT)],
                        out_hbm.at[cid, pl.ds(sid * RPT, RPT)])

    return k


def kernel(x, edge_index, distance_std, W1, b1, att1, We1, bias1,
           W2, b2, att2, We2, bias2):
    N = x.shape[0]
    E = edge_index.shape[1]
    loop = jnp.arange(N, dtype=jnp.int32)
    src = jnp.concatenate([edge_index[0].astype(jnp.int32), loop])
    dst = jnp.concatenate([edge_index[1].astype(jnp.int32), loop])
    ea = jnp.concatenate(
        [distance_std[:, 0], jnp.full((N,), 1.0, jnp.float32)])
    etot = E + N
    e_pad = -(-etot // (NC * NS * C)) * (NC * NS * C)
    pad = e_pad - etot
    src = jnp.pad(src, (0, pad), constant_values=N)
    dst = jnp.pad(dst, (0, pad), constant_values=N)
    ea = jnp.pad(ea, (0, pad))
    xpad = jnp.pad(x, ((0, ROWS - N), (0, 0)))

    xl1 = _tc_matmul_split(xpad, W1, b1)
    init = jnp.zeros((ROWS, 80), jnp.float32)
    acc1 = _sc_edge_pass(4, 1, e_pad, True)(
        src, dst, ea, xl1, att1.reshape(2, 4, 16), We1.reshape(2, 4, 16), init)

    rmat = jnp.repeat(jnp.eye(8, dtype=jnp.float32), 16, axis=1)
    xl2 = _tc_combine1(acc1, bias1, rmat, W2, b2)

    acc2 = _sc_edge_pass(1, 4, e_pad, False)(
        src, dst, ea, xl2, att2.reshape(4, 16), We2.reshape(4, 16), init)

    out = _tc_final(acc2, bias2)
    return out[:N]


# speedup vs baseline: 1.0243x; 1.0243x over previous
import functools

import jax
import jax.numpy as jnp
from jax import lax
from jax.experimental import pallas as pl
from jax.experimental.pallas import tpu as pltpu
from jax.experimental.pallas import tpu_sc as plsc

N_NODES = 10000
ROWS = 10240
C = 128
NC = 2
NS = 16


def _tc_matmul_split(x, w, b):
    R, K = x.shape
    M = w.shape[1]
    RB = R // 8

    def body(x_ref, w_ref, b_ref, o_ref):
        r = (jnp.dot(x_ref[...], w_ref[...], preferred_element_type=jnp.float32)
             + b_ref[...])
        o_ref[0] = r[:, :64]
        o_ref[1] = r[:, 64:]

    return pl.pallas_call(
        body,
        grid=(8,),
        in_specs=[
            pl.BlockSpec((RB, K), lambda i: (i, 0)),
            pl.BlockSpec((K, M), lambda i: (0, 0)),
            pl.BlockSpec((1, M), lambda i: (0, 0)),
        ],
        out_specs=pl.BlockSpec((2, RB, 64), lambda i: (0, i, 0)),
        out_shape=jax.ShapeDtypeStruct((2, R, 64), jnp.float32),
    )(x, w, b.reshape(1, M))


def _tc_combine1(acc, bias1, rmat, w2, b2):
    RB = ROWS // 8

    def body(a_ref, b1_ref, r_ref, w2_ref, b2_ref, o_ref):
        msg = jnp.concatenate([a_ref[0, :, :64], a_ref[1, :, :64]], axis=1)
        den = jnp.concatenate([a_ref[0, :, 64:68], a_ref[1, :, 64:68]], axis=1)
        denx = jnp.dot(den, r_ref[...], preferred_element_type=jnp.float32) + 1e-16
        h = jnp.maximum(msg / denx + b1_ref[...], 0.0)
        o_ref[...] = (
            jnp.dot(h, w2_ref[...], preferred_element_type=jnp.float32) + b2_ref[...]
        )

    return pl.pallas_call(
        body,
        grid=(8,),
        in_specs=[
            pl.BlockSpec((2, RB, 80), lambda i: (0, i, 0)),
            pl.BlockSpec((1, 128), lambda i: (0, 0)),
            pl.BlockSpec((8, 128), lambda i: (0, 0)),
            pl.BlockSpec((128, 64), lambda i: (0, 0)),
            pl.BlockSpec((1, 64), lambda i: (0, 0)),
        ],
        out_specs=pl.BlockSpec((RB, 64), lambda i: (i, 0)),
        out_shape=jax.ShapeDtypeStruct((ROWS, 64), jnp.float32),
    )(acc, bias1.reshape(1, 128), rmat, w2, b2.reshape(1, 64))


def _tc_final(acc, bias2):
    RB = ROWS // 8

    def body(a_ref, b2_ref, o_ref):
        a = a_ref[0] + a_ref[1]
        msg = a[:, :64]
        den = a[:, 64:65]
        o = msg / (den + 1e-16) + b2_ref[...]
        m = jnp.max(o, axis=1, keepdims=True)
        ls = m + jnp.log(jnp.sum(jnp.exp(o - m), axis=1, keepdims=True))
        o_ref[...] = o - ls

    return pl.pallas_call(
        body,
        grid=(8,),
        in_specs=[
            pl.BlockSpec((2, RB, 80), lambda i: (0, i, 0)),
            pl.BlockSpec((1, 64), lambda i: (0, 0)),
        ],
        out_specs=pl.BlockSpec((RB, 64), lambda i: (i, 0)),
        out_shape=jax.ShapeDtypeStruct((ROWS, 64), jnp.float32),
    )(acc, bias2.reshape(1, 64))


def _sc_edge_pass(heads, grp, e_pad, head_split):
    CH = heads * grp * 16
    W = CH + 16
    n_workers = NS if head_split else NC * NS
    EPT = e_pad // n_workers
    NCHUNK = EPT // C
    RPT = ROWS // NS
    mesh = plsc.VectorSubcoreMesh(core_axis_name="c", subcore_axis_name="s")

    @functools.partial(
        pl.kernel,
        out_type=jax.ShapeDtypeStruct((NC, ROWS, W), jnp.float32),
        mesh=mesh,
        compiler_params=pltpu.CompilerParams(
            use_tc_tiling_on_sc=False, needs_layout_passes=False),
        scratch_types=[
            pltpu.VMEM((C,), jnp.int32),
            pltpu.VMEM((3, C), jnp.int32),
            pltpu.VMEM((2, C + 16), jnp.float32),
            pltpu.VMEM((2, C, CH), jnp.float32),
            pltpu.VMEM((2, C, CH), jnp.float32),
            pltpu.VMEM((2, C, W), jnp.float32),
            pltpu.VMEM((heads * grp, 16), jnp.float32),
            pltpu.VMEM((heads * grp, 16), jnp.float32),
            pltpu.VMEM_SHARED((ROWS, W), jnp.float32),
            pltpu.SemaphoreType.DMA,
            pltpu.SemaphoreType.DMA,
            pltpu.SemaphoreType.DMA,
            pltpu.SemaphoreType.DMA,
        ],
    )
    def k(src_hbm, dst_hbm, ea_hbm, xl_hbm, att_hbm, we_hbm, init_hbm, out_hbm,
          src_v, dst_v, ea_v, xs_v, xd_v, msg_v, att_v, we_v, acc_sh,
          sem1, sem2, sem3, sem4):
        cid = lax.axis_index("c")
        sid = lax.axis_index("s")
        if head_split:
            wid = sid
            tbl = xl_hbm.at[cid]
            pltpu.sync_copy(att_hbm.at[cid], att_v)
            pltpu.sync_copy(we_hbm.at[cid], we_v)
        else:
            wid = cid * NS + sid
            tbl = xl_hbm
            pltpu.sync_copy(att_hbm, att_v)
            pltpu.sync_copy(we_hbm, we_v)
        pltpu.sync_copy(init_hbm.at[pl.ds(sid * RPT, RPT)],
                        acc_sh.at[pl.ds(sid * RPT, RPT)])
        lane = lax.iota(jnp.int32, 16)
        att_vals = [att_v[i, :] for i in range(heads * grp)]
        we_vals = [we_v[i, :] for i in range(heads * grp)]
        plsc.subcore_barrier()
        base_w = wid * EPT

        def chunk_body(kk, carry):
            q = kk % 2
            p = 1 - q
            r3 = lax.rem(kk, 3)
            r3p = lax.rem(kk + 2, 3)
            cps = []

            @pl.when(jnp.logical_and(kk >= 3, p == 0))
            def _wait_scatter0():
                pltpu.make_async_copy(
                    msg_v.at[0], acc_sh.at[dst_v.at[r3]], sem3).wait()

            @pl.when(jnp.logical_and(kk >= 3, p == 1))
            def _wait_scatter1():
                pltpu.make_async_copy(
                    msg_v.at[1], acc_sh.at[dst_v.at[r3]], sem4).wait()

            @pl.when(kk < NCHUNK)
            def _issue():
                base = base_w + kk * C
                pltpu.sync_copy(src_hbm.at[pl.ds(base, C)], src_v)
                pltpu.sync_copy(dst_hbm.at[pl.ds(base, C)], dst_v.at[r3])
                pltpu.sync_copy(ea_hbm.at[pl.ds(base, C)],
                                ea_v.at[q, pl.ds(0, C)])
                cps.append(pltpu.async_copy(tbl.at[src_v], xs_v.at[q], sem1))
                cps.append(pltpu.async_copy(tbl.at[dst_v.at[r3]], xd_v.at[q], sem2))

            @pl.when(kk > 0)
            def _compute():
                @plsc.parallel_loop(0, C, step=1, unroll=4)
                def edge_body(e):
                    ea_s = ea_v[p, pl.ds(e, 16)][0]
                    den = jnp.zeros((16,), jnp.float32)
                    for h in range(heads):
                        t = jnp.zeros((16,), jnp.float32)
                        xsl = []
                        for g in range(grp):
                            j = (h * grp + g) * 16
                            xs = xs_v[p, e, pl.ds(j, 16)]
                            xd = xd_v[p, e, pl.ds(j, 16)]
                            m = xs + xd + ea_s * we_vals[h * grp + g]
                            m = jnp.maximum(m, 0.2 * m)
                            t = t + m * att_vals[h * grp + g]
                            xsl.append(xs)
                        pv = jnp.exp(
                            jnp.full((16,), plsc.cumsum(t)[15], jnp.float32))
                        for g in range(grp):
                            msg_v[p, e, pl.ds((h * grp + g) * 16, 16)] = xsl[g] * pv
                        den = den + jnp.where(lane == h, pv, 0.0)
                    msg_v[p, e, pl.ds(CH, 16)] = den

                @pl.when(p == 0)
                def _scatter0():
                    pltpu.async_copy(
                        msg_v.at[0], acc_sh.at[dst_v.at[r3p]], sem3, add=True)

                @pl.when(p == 1)
                def _scatter1():
                    pltpu.async_copy(
                        msg_v.at[1], acc_sh.at[dst_v.at[r3p]], sem4, add=True)

            @pl.when(kk < NCHUNK)
            def _drain():
                for cp in cps:
                    cp.wait()

            return carry

        lax.fori_loop(0, NCHUNK + 1, chunk_body, 0)
        for chunk in (NCHUNK - 2, NCHUNK - 1):
            pltpu.make_async_copy(
                msg_v.at[chunk % 2],
                acc_sh.at[dst_v.at[chunk % 3]],
                sem3 if chunk % 2 == 0 else sem4).wait()
        plsc.subcore_barrier()
        pltpu.sync_copy(acc_sh.at[pl.ds(sid * RPT, RPT)],
                        out_hbm.at[cid, pl.ds(sid * RPT, RPT)])

    return k


def kernel(x, edge_index, distance_std, W1, b1, att1, We1, bias1,
           W2, b2, att2, We2, bias2):
    N = x.shape[0]
    E = edge_index.shape[1]
    loop = jnp.arange(N, dtype=jnp.int32)
    src = jnp.concatenate([edge_index[0].astype(jnp.int32), loop])
    dst = jnp.concatenate([edge_index[1].astype(jnp.int32), loop])
    ea = jnp.concatenate(
        [distance_std[:, 0], jnp.full((N,), 1.0, jnp.float32)])
    etot = E + N
    e_pad = -(-etot // (NC * NS * C)) * (NC * NS * C)
    pad = e_pad - etot
    src = jnp.pad(src, (0, pad), constant_values=N)
    dst = jnp.pad(dst, (0, pad), constant_values=N)
    ea = jnp.pad(ea, (0, pad))
    xpad = jnp.pad(x, ((0, ROWS - N), (0, 0)))

    xl1 = _tc_matmul_split(xpad, W1, b1)
    init = jnp.zeros((ROWS, 80), jnp.float32)
    acc1 = _sc_edge_pass(4, 1, e_pad, True)(
        src, dst, ea, xl1, att1.reshape(2, 4, 16), We1.reshape(2, 4, 16), init)

    rmat = jnp.repeat(jnp.eye(8, dtype=jnp.float32), 16, axis=1)
    xl2 = _tc_combine1(acc1, bias1, rmat, W2, b2)

    acc2 = _sc_edge_pass(1, 4, e_pad, False)(
        src, dst, ea, xl2, att2.reshape(4, 16), We2.reshape(4, 16), init)

    out = _tc_final(acc2, bias2)
    return out[:N]
